# baseline (speedup 1.0000x reference)
import functools
import math

import jax
import jax.numpy as jnp
from jax import lax
from jax.experimental import pallas as pl
from jax.experimental.pallas import tpu as pltpu
from jax.experimental.pallas import tpu_sc as plsc

MODEL_DIM = 64
SCALE = math.sqrt(MODEL_DIM)

NC = 2
NS = 16
NW = NC * NS
LANES = 16

ROWS_PER_GATHER = 128
G = 4
CHUNK = ROWS_PER_GATHER * G


def _emb_body(n_chunks, x_hbm, table_hbm, out_hbm,
              idx_v, rows_v, g0, g1, s0, s1):
  D = MODEL_DIM
  wid = lax.axis_index("s") * NC + lax.axis_index("c")
  xrow0 = wid * (n_chunks * G)
  obase = wid * (n_chunks * CHUNK)

  gsem = (g0, g1)
  ssem = (s0, s1)

  def load_idx(c, b):
    pltpu.sync_copy(x_hbm.at[pl.ds(xrow0 + c * G, G)], idx_v.at[b])

  def start_gather(b):
    for g in range(G):
      pltpu.make_async_copy(
          table_hbm.at[idx_v.at[b, g]],
          rows_v.at[b, pl.ds(g * ROWS_PER_GATHER, ROWS_PER_GATHER)],
          gsem[b]).start()

  def wait_gather(b):
    pltpu.make_async_copy(
        table_hbm.at[pl.ds(0, CHUNK)], rows_v.at[b], gsem[b]).wait()

  def scale(b):
    @pl.loop(0, CHUNK, unroll=4)
    def _(j):
      for k in range(D // LANES):
        v = rows_v[b, j, pl.ds(k * LANES, LANES)]
        rows_v[b, j, pl.ds(k * LANES, LANES)] = v * SCALE

  def start_store(c, b):
    pltpu.make_async_copy(
        rows_v.at[b], out_hbm.at[pl.ds(obase + c * CHUNK, CHUNK)],
        ssem[b]).start()

  def wait_store(b):
    pltpu.make_async_copy(
        rows_v.at[b], out_hbm.at[pl.ds(obase, CHUNK)], ssem[b]).wait()

  load_idx(0, 0)
  start_gather(0)
  load_idx(1, 1)
  start_gather(1)
  wait_gather(0)
  scale(0)
  start_store(0, 0)

  @pl.loop(1, n_chunks - 1, step=2)
  def _(c):
    for b, off in ((1, 0), (0, 1)):
      cc = c + off
      wait_store(1 - b)
      load_idx(cc + 1, 1 - b)
      start_gather(1 - b)
      wait_gather(b)
      scale(b)
      start_store(cc, b)

  last = n_chunks - 1
  wait_gather(1)
  scale(1)
  start_store(last, 1)
  wait_store(0)
  wait_store(1)


@jax.jit
def _emb_lookup(x2d, table):
  B = x2d.shape[0] * x2d.shape[1]
  D = MODEL_DIM
  n_chunks = B // (NW * CHUNK)
  xv = x2d.reshape(B // ROWS_PER_GATHER, ROWS_PER_GATHER)

  kern = pl.kernel(
      functools.partial(_emb_body, n_chunks),
      out_type=jax.ShapeDtypeStruct((B, D), jnp.float32),
      mesh=plsc.VectorSubcoreMesh(core_axis_name="c", subcore_axis_name="s"),
      compiler_params=pltpu.CompilerParams(use_tc_tiling_on_sc=False),
      scratch_types=[
          pltpu.VMEM((2, G, ROWS_PER_GATHER), jnp.int32),
          pltpu.VMEM((2, CHUNK, D), jnp.float32),
          pltpu.SemaphoreType.DMA,
          pltpu.SemaphoreType.DMA,
          pltpu.SemaphoreType.DMA,
          pltpu.SemaphoreType.DMA,
      ],
  )
  return kern(xv, table)


def kernel(x, table):
  out = _emb_lookup(x.astype(jnp.int32), table)
  return out.reshape(x.shape[0], x.shape[1], MODEL_DIM)
